# baseline (speedup 1.0000x reference)
import functools

import jax
import jax.numpy as jnp
from jax import lax
from jax.experimental import pallas as pl
from jax.experimental.pallas import tpu as pltpu
from jax.experimental.pallas import tpu_sc as plsc

_CHUNK = 128


@functools.cache
def _build(n_user, n_item, batch, d):
    info = plsc.get_sparse_core_info()
    nw = info.num_cores * info.num_subcores
    nc = info.num_cores
    b_per_w = batch // nw
    n_chunks = b_per_w // _CHUNK
    mesh = plsc.VectorSubcoreMesh(core_axis_name="c", subcore_axis_name="s")

    @functools.partial(
        pl.kernel,
        mesh=mesh,
        out_type=jax.ShapeDtypeStruct((d, 2 * batch), jnp.float32),
        compiler_params=pltpu.CompilerParams(use_tc_tiling_on_sc=False),
        scratch_types=[
            pltpu.VMEM((n_chunks, _CHUNK), jnp.int32),
            pltpu.VMEM((n_chunks, _CHUNK), jnp.int32),
            pltpu.VMEM((d, b_per_w), jnp.float32),
            pltpu.VMEM((d, b_per_w), jnp.float32),
            pltpu.SemaphoreType.DMA,
            pltpu.SemaphoreType.DMA,
        ],
    )
    def gather_kernel(user_hbm, item_hbm, idx_u_hbm, idx_i_hbm, out_hbm,
                      idx_uv, idx_iv, buf_u, buf_i, sem_u, sem_i):
        wid = lax.axis_index("s") * nc + lax.axis_index("c")
        base = wid * b_per_w
        crow = wid * n_chunks
        pltpu.sync_copy(idx_u_hbm.at[pl.ds(crow, n_chunks)], idx_uv)
        pltpu.sync_copy(idx_i_hbm.at[pl.ds(crow, n_chunks)], idx_iv)
        for c in range(d):
            for j in range(n_chunks):
                pltpu.async_copy(
                    user_hbm.at[c].at[idx_uv.at[j]],
                    buf_u.at[c, pl.ds(j * _CHUNK, _CHUNK)], sem_u)
        for c in range(d):
            for j in range(n_chunks):
                pltpu.async_copy(
                    item_hbm.at[c].at[idx_iv.at[j]],
                    buf_i.at[c, pl.ds(j * _CHUNK, _CHUNK)], sem_i)
        pltpu.make_async_copy(
            user_hbm.at[:, pl.ds(0, b_per_w)], buf_u, sem_u).wait()
        pltpu.sync_copy(buf_u, out_hbm.at[:, pl.ds(base, b_per_w)])
        pltpu.make_async_copy(
            item_hbm.at[:, pl.ds(0, b_per_w)], buf_i, sem_i).wait()
        pltpu.sync_copy(buf_i, out_hbm.at[:, pl.ds(batch + base, b_per_w)])

    return gather_kernel


@jax.jit
def kernel(embed_user, embed_item, idx_user, idx_item):
    batch = idx_user.shape[0]
    d = embed_user.shape[1]
    idx_u2 = idx_user.astype(jnp.int32).reshape(batch // _CHUNK, _CHUNK)
    idx_i2 = idx_item.astype(jnp.int32).reshape(batch // _CHUNK, _CHUNK)
    k = _build(embed_user.shape[0], embed_item.shape[0], batch, d)
    out_t = k(embed_user.T, embed_item.T, idx_u2, idx_i2)
    return out_t.T


# speedup vs baseline: 7.5700x; 7.5700x over previous
import functools

import jax
import jax.numpy as jnp
from jax import lax
from jax.experimental import pallas as pl
from jax.experimental.pallas import tpu as pltpu
from jax.experimental.pallas import tpu_sc as plsc

_CHUNK = 128


@functools.cache
def _build(n_user, n_item, batch, d):
    info = plsc.get_sparse_core_info()
    nw = info.num_cores * info.num_subcores
    nc = info.num_cores
    b_per_w = batch // nw
    n_chunks = b_per_w // _CHUNK
    mesh = plsc.VectorSubcoreMesh(core_axis_name="c", subcore_axis_name="s")

    @functools.partial(
        pl.kernel,
        mesh=mesh,
        out_type=jax.ShapeDtypeStruct((2 * batch, d), jnp.float32),
        compiler_params=pltpu.CompilerParams(use_tc_tiling_on_sc=False),
        scratch_types=[
            pltpu.VMEM((n_chunks, _CHUNK), jnp.int32),
            pltpu.VMEM((n_chunks, _CHUNK), jnp.int32),
            pltpu.VMEM((b_per_w, d), jnp.float32),
            pltpu.VMEM((b_per_w, d), jnp.float32),
            pltpu.SemaphoreType.DMA,
            pltpu.SemaphoreType.DMA,
        ],
    )
    def gather_kernel(user_hbm, item_hbm, idx_u_hbm, idx_i_hbm, out_hbm,
                      idx_uv, idx_iv, rows_uv, rows_iv, sem_u, sem_i):
        wid = lax.axis_index("s") * nc + lax.axis_index("c")
        crow = wid * n_chunks
        pltpu.sync_copy(idx_u_hbm.at[pl.ds(crow, n_chunks)], idx_uv)
        pltpu.sync_copy(idx_i_hbm.at[pl.ds(crow, n_chunks)], idx_iv)
        u_copies = [
            pltpu.async_copy(user_hbm.at[idx_uv.at[j]],
                             rows_uv.at[pl.ds(j * _CHUNK, _CHUNK)], sem_u)
            for j in range(n_chunks)
        ]
        i_copies = [
            pltpu.async_copy(item_hbm.at[idx_iv.at[j]],
                             rows_iv.at[pl.ds(j * _CHUNK, _CHUNK)], sem_i)
            for j in range(n_chunks)
        ]
        base = wid * b_per_w
        for c in u_copies:
            c.wait()
        pltpu.sync_copy(rows_uv, out_hbm.at[pl.ds(base, b_per_w)])
        for c in i_copies:
            c.wait()
        pltpu.sync_copy(rows_iv, out_hbm.at[pl.ds(batch + base, b_per_w)])

    return gather_kernel


@jax.jit
def kernel(embed_user, embed_item, idx_user, idx_item):
    batch = idx_user.shape[0]
    d = embed_user.shape[1]
    idx_u2 = idx_user.astype(jnp.int32).reshape(batch // _CHUNK, _CHUNK)
    idx_i2 = idx_item.astype(jnp.int32).reshape(batch // _CHUNK, _CHUNK)
    k = _build(embed_user.shape[0], embed_item.shape[0], batch, d)
    return k(embed_user, embed_item, idx_u2, idx_i2)
